# baseline (speedup 1.0000x reference)
import jax
import jax.numpy as jnp
from jax.experimental import pallas as pl
from jax.experimental.pallas import tpu as pltpu

_T, _V, _F, _H = 50, 64, 8, 64
_TB = 2


def _elu(x):
    return jnp.where(x > 0, x, jnp.exp(jnp.minimum(x, 0.0)) - 1.0)


def _body(x_ref, w1a, b1a, w1b, b1b, w2as, w2ar, b2a, w2b, b2b,
          w3a, b3a, w3b, b3b, w4as, w4ar, w4ak, b4a, w4b, b4b, out_ref):
    f32 = jnp.float32
    dot = lambda a, b: jax.lax.dot(a, b, preferred_element_type=f32)

    x = x_ref[...]
    x1 = _elu(dot(x, w1a[...]) + b1a[...])
    x1 = _elu(dot(x1, w1b[...]) + b1b[...])

    a2 = (dot(x1, w2as[...]) + b2a[...]).reshape(_TB, _V, 1, _H)
    b2 = dot(x1, w2ar[...]).reshape(_TB, 1, _V, _H)
    h2 = _elu(a2 + b2)
    x2 = _elu(dot(h2.reshape(_TB * _V * _V, _H), w2b[...]) + b2b[...])
    g2 = x2.reshape(_TB, _V, _V, _H)

    s_ids = jax.lax.broadcasted_iota(jnp.int32, (_TB, _V, _V, _H), 1)
    r_ids = jax.lax.broadcasted_iota(jnp.int32, (_TB, _V, _V, _H), 2)
    masked = jnp.where(s_ids != r_ids, g2, 0.0)
    agg = jnp.sum(masked, axis=1).reshape(_TB * _V, _H)

    x3 = _elu(dot(agg, w3a[...]) + b3a[...])
    x3 = _elu(dot(x3, w3b[...]) + b3b[...])

    c4 = (dot(x3, w4as[...]) + b4a[...]).reshape(_TB, _V, 1, _H)
    d4 = dot(x3, w4ar[...]).reshape(_TB, 1, _V, _H)
    e4 = dot(x2, w4ak[...]).reshape(_TB, _V, _V, _H)
    h4 = _elu(c4 + d4 + e4)
    o = _elu(dot(h4.reshape(_TB * _V * _V, _H), w4b[...]) + b4b[...])
    o = o.reshape(_TB, _V, _V, _H)

    jj = jax.lax.broadcasted_iota(jnp.int32, (_TB, _V, _V - 1, _H), 2)
    ss = jax.lax.broadcasted_iota(jnp.int32, (_TB, _V, _V - 1, _H), 1)
    out = jnp.where(jj < ss, o[:, :, :_V - 1, :], o[:, :, 1:, :])
    out_ref[...] = out.reshape(_TB * _V * (_V - 1), _H)


def kernel(inputs, node_masks, all_node_inds, all_graph_info,
           w1a, b1a, w1b, b1b, w2a, b2a, w2b, b2b,
           w3a, b3a, w3b, b3b, w4a, b4a, w4b, b4b):
    b, t, v, f = inputs.shape
    h = w1b.shape[-1]
    x = inputs.reshape(t * v, f) * node_masks.reshape(t * v, 1)

    row = lambda z: z.reshape(1, h)
    wspec = lambda s: pl.BlockSpec(s, lambda i: (0, 0))
    args = [
        x,
        w1a, row(b1a), w1b, row(b1b),
        w2a[:h], w2a[h:], row(b2a), w2b, row(b2b),
        w3a, row(b3a), w3b, row(b3b),
        w4a[:h], w4a[h:2 * h], w4a[2 * h:], row(b4a), w4b, row(b4b),
    ]
    in_specs = [pl.BlockSpec((_TB * v, f), lambda i: (i, 0))]
    in_specs += [wspec(a.shape) for a in args[1:]]

    return pl.pallas_call(
        _body,
        grid=(t // _TB,),
        in_specs=in_specs,
        out_specs=pl.BlockSpec((_TB * v * (v - 1), h), lambda i: (i, 0)),
        out_shape=jax.ShapeDtypeStruct((t * v * (v - 1), h), jnp.float32),
        compiler_params=pltpu.CompilerParams(
            dimension_semantics=("arbitrary",),
        ),
    )(*args)


# speedup vs baseline: 1.0868x; 1.0868x over previous
import jax
import jax.numpy as jnp
from jax.experimental import pallas as pl
from jax.experimental.pallas import tpu as pltpu

_T, _V, _F, _H = 50, 64, 8, 64
_TB = 5


def _elu(x):
    return jnp.where(x > 0, x, jnp.exp(x) - 1.0)


def _body(x_ref, w1a, b1a, w1b, b1b, w2as, w2ar, b2a, w2b, b2b,
          w3a, b3a, w3b, b3b, w4as, w4ar, w4ak, b4a, w4b, b4b, out_ref):
    f32 = jnp.float32
    dot = lambda a, b: jax.lax.dot(a, b, preferred_element_type=f32)

    x = x_ref[...]
    x1 = _elu(dot(x, w1a[...]) + b1a[...])
    x1 = _elu(dot(x1, w1b[...]) + b1b[...])

    a2 = (dot(x1, w2as[...]) + b2a[...]).reshape(_TB, _V, 1, _H)
    b2 = dot(x1, w2ar[...]).reshape(_TB, 1, _V, _H)
    h2 = _elu(a2 + b2)
    x2 = _elu(dot(h2.reshape(_TB * _V * _V, _H), w2b[...]) + b2b[...])
    g2 = x2.reshape(_TB, _V, _V, _H)

    s_ids = jax.lax.broadcasted_iota(jnp.int32, (_TB, _V, _V, _H), 1)
    r_ids = jax.lax.broadcasted_iota(jnp.int32, (_TB, _V, _V, _H), 2)
    masked = jnp.where(s_ids != r_ids, g2, 0.0)
    agg = jnp.sum(masked, axis=1).reshape(_TB * _V, _H)

    x3 = _elu(dot(agg, w3a[...]) + b3a[...])
    x3 = _elu(dot(x3, w3b[...]) + b3b[...])

    c4 = (dot(x3, w4as[...]) + b4a[...]).reshape(_TB, _V, 1, _H)
    d4 = dot(x3, w4ar[...]).reshape(_TB, 1, _V, _H)
    e4 = dot(x2, w4ak[...]).reshape(_TB, _V, _V, _H)
    h4 = _elu(c4 + d4 + e4)
    o = _elu(dot(h4.reshape(_TB * _V * _V, _H), w4b[...]) + b4b[...])
    o = o.reshape(_TB, _V, _V, _H)

    jj = jax.lax.broadcasted_iota(jnp.int32, (_TB, _V, _V - 1, _H), 2)
    ss = jax.lax.broadcasted_iota(jnp.int32, (_TB, _V, _V - 1, _H), 1)
    out = jnp.where(jj < ss, o[:, :, :_V - 1, :], o[:, :, 1:, :])
    out_ref[...] = out.reshape(_TB * _V * (_V - 1), _H)


def kernel(inputs, node_masks, all_node_inds, all_graph_info,
           w1a, b1a, w1b, b1b, w2a, b2a, w2b, b2b,
           w3a, b3a, w3b, b3b, w4a, b4a, w4b, b4b):
    b, t, v, f = inputs.shape
    h = w1b.shape[-1]
    x = inputs.reshape(t * v, f) * node_masks.reshape(t * v, 1)

    row = lambda z: z.reshape(1, h)
    wspec = lambda s: pl.BlockSpec(s, lambda i: (0, 0))
    args = [
        x,
        w1a, row(b1a), w1b, row(b1b),
        w2a[:h], w2a[h:], row(b2a), w2b, row(b2b),
        w3a, row(b3a), w3b, row(b3b),
        w4a[:h], w4a[h:2 * h], w4a[2 * h:], row(b4a), w4b, row(b4b),
    ]
    in_specs = [pl.BlockSpec((_TB * v, f), lambda i: (i, 0))]
    in_specs += [wspec(a.shape) for a in args[1:]]

    return pl.pallas_call(
        _body,
        grid=(t // _TB,),
        in_specs=in_specs,
        out_specs=pl.BlockSpec((_TB * v * (v - 1), h), lambda i: (i, 0)),
        out_shape=jax.ShapeDtypeStruct((t * v * (v - 1), h), jnp.float32),
        compiler_params=pltpu.CompilerParams(
            dimension_semantics=("arbitrary",),
        ),
    )(*args)
